# baseline (speedup 1.0000x reference)
import functools

import jax
import jax.numpy as jnp
from jax import lax
from jax.experimental import pallas as pl
from jax.experimental.pallas import tpu as pltpu
from jax.experimental.pallas import tpu_sc as plsc

B = 16384
D = 128
NC, NS = 2, 16
NW = NC * NS
NCHUNK = 1
CB = B // NCHUNK
BPW = CB // NW
TC_BLK = 4096


NB = 3
PF = 2
NSPLIT = 2
STEP = BPW // NSPLIT


def _sc_gather_body(head_hbm, tail_hbm, rg_hbm, ap_hbm, ent_hbm, gn_hbm,
                    out_hbm, hidx, tidx, *scratch):
    bufs = scratch[:NB]
    gsems = scratch[NB:2 * NB]
    wsems = scratch[2 * NB:3 * NB]
    wid = lax.axis_index("s") * NC + lax.axis_index("c")
    base = wid * BPW
    pltpu.sync_copy(head_hbm.at[pl.ds(base, BPW)], hidx)
    pltpu.sync_copy(tail_hbm.at[pl.ds(base, BPW)], tidx)
    plan = [(rg_hbm, hidx, 0), (rg_hbm, tidx, 1),
            (ap_hbm, hidx, 2), (ap_hbm, tidx, 3),
            (ent_hbm, hidx, 4), (ent_hbm, tidx, 5),
            (gn_hbm, hidx, 6), (gn_hbm, tidx, 7)]
    steps = [(tab, idx, col, h)
             for tab, idx, col in plan for h in range(NSPLIT)]
    n = len(steps)

    def gstart(k):
        tab, idx, _, h = steps[k]
        b = k % NB
        return pltpu.async_copy(tab.at[idx.at[pl.ds(h * STEP, STEP)]],
                                bufs[b], gsems[b])

    def wstart(k):
        _, _, col, h = steps[k]
        b = k % NB
        return pltpu.async_copy(
            bufs[b],
            out_hbm.at[col, pl.ds(base + h * STEP, STEP)],
            wsems[b])

    gh = [None] * n
    wh = [None] * n
    for k in range(PF):
        gh[k] = gstart(k)
    for k in range(n):
        j = k + PF
        if j < n and j >= NB:
            wh[j - NB].wait()
        if j < n:
            gh[j] = gstart(j)
        gh[k].wait()
        wh[k] = wstart(k)
    for k in range(n - NB, n):
        wh[k].wait()


def _sc_gather(head, tail, rg, ap, ent, gn):
    mesh = plsc.VectorSubcoreMesh(core_axis_name="c", subcore_axis_name="s",
                                  num_cores=NC, num_subcores=NS)
    fn = functools.partial(
        pl.kernel, mesh=mesh,
        out_type=jax.ShapeDtypeStruct((8, CB, D), jnp.float32),
        scratch_types=(
            [pltpu.VMEM((BPW,), jnp.int32),
             pltpu.VMEM((BPW,), jnp.int32)]
            + [pltpu.VMEM((STEP, D), jnp.float32)] * NB
            + [pltpu.SemaphoreType.DMA] * (2 * NB)
        ),
    )(_sc_gather_body)
    return fn(head, tail, rg, ap, ent, gn)


def _tc_mlp_body(rel_ref, g_ref, rel_emb_ref, w1_ref, b1_ref, w2_ref,
                 b2_ref, out_ref):
    rel_row = rel_ref[0]
    onehot_t = (lax.broadcasted_iota(jnp.int32, (64, TC_BLK), 0)
                == rel_row).astype(jnp.float32)
    r_e = lax.dot_general(onehot_t, rel_emb_ref[...],
                          (((0,), (0,)), ((), ())),
                          preferred_element_type=jnp.float32)
    s = jax.nn.sigmoid(g_ref[4] + r_e - g_ref[5])
    hid = (jnp.dot(s, w1_ref[4 * D:5 * D, :],
                   preferred_element_type=jnp.float32)
           + b1_ref[...])
    for k, w0 in ((0, 0), (1, 1), (2, 2), (3, 3), (6, 5), (7, 6)):
        hid += jnp.dot(g_ref[k], w1_ref[w0 * D:(w0 + 1) * D, :],
                       preferred_element_type=jnp.float32)
    hid = jnp.maximum(hid, 0.0)
    logits_t = lax.dot_general(w2_ref[...], hid,
                               (((0,), (1,)), ((), ())),
                               preferred_element_type=jnp.float32)
    logits_t += b2_ref[...]
    m = jnp.max(logits_t, axis=0, keepdims=True)
    e = jnp.exp(logits_t - m)
    out_ref[...] = e / jnp.sum(e, axis=0, keepdims=True)


def _tc_mlp(rel3d, g, rel_emb, w1, b1, w2, b2):
    nblk = CB // TC_BLK
    return pl.pallas_call(
        _tc_mlp_body,
        grid=(nblk,),
        in_specs=[
            pl.BlockSpec((1, 1, TC_BLK), lambda i: (i, 0, 0)),
            pl.BlockSpec((8, TC_BLK, D), lambda i: (0, i, 0)),
            pl.BlockSpec((64, D), lambda i: (0, 0)),
            pl.BlockSpec((7 * D, D), lambda i: (0, 0)),
            pl.BlockSpec((1, D), lambda i: (0, 0)),
            pl.BlockSpec((D, 2), lambda i: (0, 0)),
            pl.BlockSpec((2, 1), lambda i: (0, 0)),
        ],
        out_specs=pl.BlockSpec((2, TC_BLK), lambda i: (0, i)),
        out_shape=jax.ShapeDtypeStruct((2, CB), jnp.float32),
    )(rel3d, g, rel_emb, w1, b1, w2, b2)


def kernel(head, rel, tail, ent_emb, rel_emb, rg_feature, ap_feature,
           gn_feature, W1, b1, W2, b2):
    g = _sc_gather(head, tail, rg_feature, ap_feature, ent_emb, gn_feature)
    out_t = _tc_mlp(rel.reshape(B // TC_BLK, 1, TC_BLK), g, rel_emb, W1,
                    b1.reshape(1, D), W2, b2.reshape(2, 1))
    return out_t.T


# speedup vs baseline: 1.0149x; 1.0149x over previous
import functools

import jax
import jax.numpy as jnp
from jax import lax
from jax.experimental import pallas as pl
from jax.experimental.pallas import tpu as pltpu
from jax.experimental.pallas import tpu_sc as plsc

B = 16384
D = 128
NC, NS = 2, 16
NW = NC * NS
NCHUNK = 1
CB = B // NCHUNK
BPW = CB // NW
TC_BLK = 2048


NB = 3
PF = 2
NSPLIT = 2
STEP = BPW // NSPLIT


def _sc_gather_body(head_hbm, tail_hbm, rg_hbm, ap_hbm, ent_hbm, gn_hbm,
                    out_hbm, hidx, tidx, *scratch):
    bufs = scratch[:NB]
    gsems = scratch[NB:2 * NB]
    wsems = scratch[2 * NB:3 * NB]
    wid = lax.axis_index("s") * NC + lax.axis_index("c")
    base = wid * BPW
    pltpu.sync_copy(head_hbm.at[pl.ds(base, BPW)], hidx)
    pltpu.sync_copy(tail_hbm.at[pl.ds(base, BPW)], tidx)
    plan = [(rg_hbm, hidx, 0), (rg_hbm, tidx, 1),
            (ap_hbm, hidx, 2), (ap_hbm, tidx, 3),
            (ent_hbm, hidx, 4), (ent_hbm, tidx, 5),
            (gn_hbm, hidx, 6), (gn_hbm, tidx, 7)]
    steps = [(tab, idx, col, h)
             for tab, idx, col in plan for h in range(NSPLIT)]
    n = len(steps)

    def gstart(k):
        tab, idx, _, h = steps[k]
        b = k % NB
        return pltpu.async_copy(tab.at[idx.at[pl.ds(h * STEP, STEP)]],
                                bufs[b], gsems[b])

    def wstart(k):
        _, _, col, h = steps[k]
        b = k % NB
        return pltpu.async_copy(
            bufs[b],
            out_hbm.at[col, pl.ds(base + h * STEP, STEP)],
            wsems[b])

    gh = [None] * n
    wh = [None] * n
    for k in range(PF):
        gh[k] = gstart(k)
    for k in range(n):
        j = k + PF
        if j < n and j >= NB:
            wh[j - NB].wait()
        if j < n:
            gh[j] = gstart(j)
        gh[k].wait()
        wh[k] = wstart(k)
    for k in range(n - NB, n):
        wh[k].wait()


def _sc_gather(head, tail, rg, ap, ent, gn):
    mesh = plsc.VectorSubcoreMesh(core_axis_name="c", subcore_axis_name="s",
                                  num_cores=NC, num_subcores=NS)
    fn = functools.partial(
        pl.kernel, mesh=mesh,
        out_type=jax.ShapeDtypeStruct((8, CB, D), jnp.float32),
        scratch_types=(
            [pltpu.VMEM((BPW,), jnp.int32),
             pltpu.VMEM((BPW,), jnp.int32)]
            + [pltpu.VMEM((STEP, D), jnp.float32)] * NB
            + [pltpu.SemaphoreType.DMA] * (2 * NB)
        ),
    )(_sc_gather_body)
    return fn(head, tail, rg, ap, ent, gn)


def _tc_mlp_body(rel_ref, g_ref, rel_emb_ref, w1_ref, b1_ref, w2_ref,
                 b2_ref, out_ref):
    rel_row = rel_ref[0]
    onehot_t = (lax.broadcasted_iota(jnp.int32, (64, TC_BLK), 0)
                == rel_row).astype(jnp.float32)
    r_e = lax.dot_general(onehot_t, rel_emb_ref[...],
                          (((0,), (0,)), ((), ())),
                          preferred_element_type=jnp.float32)
    s = jax.nn.sigmoid(g_ref[4] + r_e - g_ref[5])
    hid = (jnp.dot(s, w1_ref[4 * D:5 * D, :],
                   preferred_element_type=jnp.float32)
           + b1_ref[...])
    for k, w0 in ((0, 0), (1, 1), (2, 2), (3, 3), (6, 5), (7, 6)):
        hid += jnp.dot(g_ref[k], w1_ref[w0 * D:(w0 + 1) * D, :],
                       preferred_element_type=jnp.float32)
    hid = jnp.maximum(hid, 0.0)
    logits_t = lax.dot_general(w2_ref[...], hid,
                               (((0,), (1,)), ((), ())),
                               preferred_element_type=jnp.float32)
    logits_t += b2_ref[...]
    m = jnp.max(logits_t, axis=0, keepdims=True)
    e = jnp.exp(logits_t - m)
    out_ref[...] = e / jnp.sum(e, axis=0, keepdims=True)


def _tc_mlp(rel3d, g, rel_emb, w1, b1, w2, b2):
    nblk = CB // TC_BLK
    return pl.pallas_call(
        _tc_mlp_body,
        grid=(nblk,),
        in_specs=[
            pl.BlockSpec((1, 1, TC_BLK), lambda i: (i, 0, 0)),
            pl.BlockSpec((8, TC_BLK, D), lambda i: (0, i, 0)),
            pl.BlockSpec((64, D), lambda i: (0, 0)),
            pl.BlockSpec((7 * D, D), lambda i: (0, 0)),
            pl.BlockSpec((1, D), lambda i: (0, 0)),
            pl.BlockSpec((D, 2), lambda i: (0, 0)),
            pl.BlockSpec((2, 1), lambda i: (0, 0)),
        ],
        out_specs=pl.BlockSpec((2, TC_BLK), lambda i: (0, i)),
        out_shape=jax.ShapeDtypeStruct((2, CB), jnp.float32),
    )(rel3d, g, rel_emb, w1, b1, w2, b2)


def kernel(head, rel, tail, ent_emb, rel_emb, rg_feature, ap_feature,
           gn_feature, W1, b1, W2, b2):
    g = _sc_gather(head, tail, rg_feature, ap_feature, ent_emb, gn_feature)
    out_t = _tc_mlp(rel.reshape(B // TC_BLK, 1, TC_BLK), g, rel_emb, W1,
                    b1.reshape(1, D), W2, b2.reshape(2, 1))
    return out_t.T
